# baseline (speedup 1.0000x reference)
import functools

import jax
import jax.numpy as jnp
from jax import lax
from jax.experimental import pallas as pl
from jax.experimental.pallas import tpu as pltpu
from jax.experimental.pallas import tpu_sc as plsc

F32 = jnp.float32

NC = 2
NS = 16
NW = NC * NS

NUM_LAYERS = 10
NEG_SLOPE = 0.01
BN_EPS = 1e-5


def _pick_chunk(epw: int) -> int:
    for c in range(128, 0, -8):
        if epw % c == 0:
            return c
    raise ValueError(f"no chunk divides {epw}")


def _pack_bf16_pair(hi_f32, lo_f32):
    hb = lax.bitcast_convert_type(hi_f32, jnp.uint32)
    lb = lax.bitcast_convert_type(lo_f32, jnp.uint32)
    hi = (hb + jnp.uint32(0x8000)) & jnp.uint32(0xFFFF0000)
    lo = (lb + jnp.uint32(0x8000)) >> jnp.uint32(16)
    return hi | lo


def _unpack_bf16_pair(p_u32):
    f = lax.bitcast_convert_type(p_u32 & jnp.uint32(0xFFFF0000), F32)
    s = lax.bitcast_convert_type(p_u32 << jnp.uint32(16), F32)
    return f, s


def _proj_body(x_ref, wd_ref, ws_ref, td_ref, ts_ref):
    xv = x_ref[...]
    f = xv.shape[1]
    td = jnp.dot(xv, wd_ref[...], preferred_element_type=F32)
    ts = jnp.dot(xv, ws_ref[...], preferred_element_type=F32)
    td_ref[...] = _pack_bf16_pair(td[:, :f], td[:, f:])
    ts_ref[...] = _pack_bf16_pair(ts[:, :f], ts[:, f:])


def _proj(x, wd, ws):
    n, f = x.shape
    f2 = wd.shape[1]
    bn = 2000
    grid = n // bn
    return pl.pallas_call(
        _proj_body,
        grid=(grid,),
        in_specs=[
            pl.BlockSpec((bn, f), lambda i: (i, 0)),
            pl.BlockSpec((f, f2), lambda i: (0, 0)),
            pl.BlockSpec((f, f2), lambda i: (0, 0)),
        ],
        out_specs=[
            pl.BlockSpec((bn, f), lambda i: (i, 0)),
            pl.BlockSpec((bn, f), lambda i: (i, 0)),
        ],
        out_shape=[
            jax.ShapeDtypeStruct((n, f), jnp.uint32),
            jax.ShapeDtypeStruct((n, f), jnp.uint32),
        ],
    )(x, wd, ws)


def _gather(td, ts, dst, src):
    e = dst.shape[0]
    f = td.shape[1]
    epw = e // NW
    chunk = _pick_chunk(epw)
    nsteps = epw // chunk
    mesh = plsc.VectorSubcoreMesh(
        core_axis_name="c", subcore_axis_name="s", num_cores=NC, num_subcores=NS
    )

    @functools.partial(
        pl.kernel,
        mesh=mesh,
        out_type=[
            jax.ShapeDtypeStruct((e, f), jnp.uint32),
            jax.ShapeDtypeStruct((e, f), jnp.uint32),
        ],
        scratch_types=[
            pltpu.VMEM((chunk,), jnp.int32),
            pltpu.VMEM((chunk,), jnp.int32),
            pltpu.VMEM((chunk,), jnp.int32),
            pltpu.VMEM((chunk,), jnp.int32),
            pltpu.VMEM((chunk, f), jnp.uint32),
            pltpu.VMEM((chunk, f), jnp.uint32),
            pltpu.VMEM((chunk, f), jnp.uint32),
            pltpu.VMEM((chunk, f), jnp.uint32),
            pltpu.SemaphoreType.DMA,
            pltpu.SemaphoreType.DMA,
            pltpu.SemaphoreType.DMA,
            pltpu.SemaphoreType.DMA,
            pltpu.SemaphoreType.DMA,
            pltpu.SemaphoreType.DMA,
        ],
    )
    def k(td_hbm, ts_hbm, dst_hbm, src_hbm, gd_hbm, gs_hbm,
          di0, di1, sri0, sri1, rd0, rd1, rs0, rs1, si0, si1, sg0, sg1,
          sw0, sw1):
        wid = lax.axis_index("s") * NC + lax.axis_index("c")
        tbase = wid * epw
        di = (di0, di1)
        sri = (sri0, sri1)
        rd = (rd0, rd1)
        rs = (rs0, rs1)
        si = (si0, si1)
        sg = (sg0, sg1)
        sw = (sw0, sw1)

        def issue_gathers(b, step):
            pltpu.async_copy(td_hbm.at[di[b]], rd[b], sg[b])
            pltpu.async_copy(ts_hbm.at[sri[b]], rs[b], sg[b])

        def wait_gathers(b, step):
            pltpu.make_async_copy(td_hbm.at[di[b]], rd[b], sg[b]).wait()
            pltpu.make_async_copy(ts_hbm.at[sri[b]], rs[b], sg[b]).wait()

        def issue_idx(b, step):
            base = tbase + step * chunk
            pltpu.async_copy(dst_hbm.at[pl.ds(base, chunk)], di[b], si[b])
            pltpu.async_copy(src_hbm.at[pl.ds(base, chunk)], sri[b], si[b])

        def wait_idx(b, step):
            base = tbase + step * chunk
            pltpu.make_async_copy(
                dst_hbm.at[pl.ds(base, chunk)], di[b], si[b]).wait()
            pltpu.make_async_copy(
                src_hbm.at[pl.ds(base, chunk)], sri[b], si[b]).wait()

        def issue_write(b, step):
            base = tbase + step * chunk
            pltpu.async_copy(rd[b], gd_hbm.at[pl.ds(base, chunk)], sw[b])
            pltpu.async_copy(rs[b], gs_hbm.at[pl.ds(base, chunk)], sw[b])

        def wait_write(b, step):
            base = tbase + step * chunk
            pltpu.make_async_copy(
                rd[b], gd_hbm.at[pl.ds(base, chunk)], sw[b]).wait()
            pltpu.make_async_copy(
                rs[b], gs_hbm.at[pl.ds(base, chunk)], sw[b]).wait()

        pltpu.sync_copy(dst_hbm.at[pl.ds(tbase, chunk)], di0)
        pltpu.sync_copy(src_hbm.at[pl.ds(tbase, chunk)], sri0)
        issue_gathers(0, 0)
        issue_idx(1, 1)

        def step_body(kk, b):
            o = 1 - b
            wait_gathers(b, kk)

            @pl.when(kk >= 1)
            def _():
                wait_write(o, kk - 1)

            @pl.when(kk + 1 < nsteps)
            def _():
                wait_idx(o, kk + 1)
                issue_gathers(o, kk + 1)

                @pl.when(kk + 2 < nsteps)
                def _():
                    issue_idx(b, kk + 2)

            issue_write(b, kk)

        def body(kk, carry):
            @pl.when(kk % 2 == 0)
            def _():
                step_body(kk, 0)

            @pl.when(kk % 2 == 1)
            def _():
                step_body(kk, 1)

            return carry

        lax.fori_loop(0, nsteps, body, 0)
        wait_write((nsteps - 1) % 2, nsteps - 1)

    return k(td, ts, dst, src)


def _message_body(gd_ref, gs_ref, ea_ref, wfe_ref, wse_ref, bf_ref, bs_ref,
                  msg_ref, *, f):
    gd_f, gd_s = _unpack_bf16_pair(gd_ref[...])
    gs_f, gs_s = _unpack_bf16_pair(gs_ref[...])
    eav = ea_ref[...]
    uf = (gd_f + gs_f
          + jnp.dot(eav, wfe_ref[...], preferred_element_type=F32) + bf_ref[...])
    us = (gd_s + gs_s
          + jnp.dot(eav, wse_ref[...], preferred_element_type=F32) + bs_ref[...])
    sig = jax.nn.sigmoid(uf)
    sp = jax.nn.softplus(us)
    msg_ref[...] = sig * sp


def _message(gd, gs, ea, wfe, wse, bfl, bsl):
    e, f = gd.shape
    ed = ea.shape[1]
    be = 2560
    grid = e // be
    return pl.pallas_call(
        functools.partial(_message_body, f=f),
        grid=(grid,),
        in_specs=[
            pl.BlockSpec((be, f), lambda i: (i, 0)),
            pl.BlockSpec((be, f), lambda i: (i, 0)),
            pl.BlockSpec((be, ed), lambda i: (i, 0)),
            pl.BlockSpec((ed, f), lambda i: (0, 0)),
            pl.BlockSpec((ed, f), lambda i: (0, 0)),
            pl.BlockSpec((1, f), lambda i: (0, 0)),
            pl.BlockSpec((1, f), lambda i: (0, 0)),
        ],
        out_specs=pl.BlockSpec((be, f), lambda i: (i, 0)),
        out_shape=jax.ShapeDtypeStruct((e, f), F32),
    )(gd, gs, ea, wfe, wse, bfl, bsl)


def _scatter(msg, dst, n, prev=None):
    e, f = msg.shape
    epw = e // NW
    chunk = _pick_chunk(epw)
    nchunks = epw // chunk
    rows_per_tile = -(-n // NS)
    rows_per_tile = ((rows_per_tile + chunk - 1) // chunk) * chunk
    npad = rows_per_tile * NS
    copies = rows_per_tile // chunk
    mesh = plsc.VectorSubcoreMesh(
        core_axis_name="c", subcore_axis_name="s", num_cores=NC, num_subcores=NS
    )
    has_prev = prev is not None

    def k_body(msg_hbm, dst_hbm, *rest):
        if has_prev:
            prev_hbm = rest[0]
            rest = rest[1:]
        (out_hbm, dsti0, dsti1, msgv0, msgv1, acc, sl0, sl1, sa0, sa1) = rest
        c = lax.axis_index("c")
        s = lax.axis_index("s")
        wid = s * NC + c
        tbase = wid * epw
        dsti = (dsti0, dsti1)
        msgv = (msgv0, msgv1)
        sl = (sl0, sl1)
        sa = (sa0, sa1)

        if has_prev:
            pltpu.sync_copy(
                prev_hbm.at[c, pl.ds(s * rows_per_tile, rows_per_tile)],
                acc.at[pl.ds(s * rows_per_tile, rows_per_tile)])
        else:
            def zrow(i, carry):
                for j in range(f // 16):
                    msgv0[i, pl.ds(j * 16, 16)] = jnp.zeros((16,), F32)
                return carry

            lax.fori_loop(0, chunk, zrow, 0)
            for j in range(copies):
                pltpu.sync_copy(
                    msgv0, acc.at[pl.ds(s * rows_per_tile + j * chunk, chunk)])
        plsc.subcore_barrier()

        def issue_loads(b, step):
            base = tbase + step * chunk
            pltpu.async_copy(dst_hbm.at[pl.ds(base, chunk)], dsti[b], sl[b])
            pltpu.async_copy(msg_hbm.at[pl.ds(base, chunk)], msgv[b], sl[b])

        def wait_loads(b, step):
            base = tbase + step * chunk
            pltpu.make_async_copy(
                dst_hbm.at[pl.ds(base, chunk)], dsti[b], sl[b]).wait()
            pltpu.make_async_copy(
                msg_hbm.at[pl.ds(base, chunk)], msgv[b], sl[b]).wait()

        issue_loads(0, 0)

        def wait_add(b):
            pltpu.make_async_copy(msgv[b], acc.at[dsti[b]], sa[b]).wait()

        def step_body(kk, b):
            o = 1 - b

            @pl.when(kk >= 1)
            def _():
                wait_add(o)

            @pl.when(kk + 1 < nchunks)
            def _():
                issue_loads(o, kk + 1)

            wait_loads(b, kk)
            pltpu.async_copy(msgv[b], acc.at[dsti[b]], sa[b], add=True)

        def body(kk, carry):
            @pl.when(kk % 2 == 0)
            def _():
                step_body(kk, 0)

            @pl.when(kk % 2 == 1)
            def _():
                step_body(kk, 1)

            return carry

        lax.fori_loop(0, nchunks, body, 0)
        wait_add((nchunks - 1) % 2)
        plsc.subcore_barrier()
        pltpu.sync_copy(
            acc.at[pl.ds(s * rows_per_tile, rows_per_tile)],
            out_hbm.at[c, pl.ds(s * rows_per_tile, rows_per_tile)])

    k = functools.partial(
        pl.kernel,
        mesh=mesh,
        out_type=jax.ShapeDtypeStruct((NC, npad, f), F32),
        scratch_types=[
            pltpu.VMEM((chunk,), jnp.int32),
            pltpu.VMEM((chunk,), jnp.int32),
            pltpu.VMEM((chunk, f), F32),
            pltpu.VMEM((chunk, f), F32),
            pltpu.VMEM_SHARED((npad, f), F32),
            pltpu.SemaphoreType.DMA,
            pltpu.SemaphoreType.DMA,
            pltpu.SemaphoreType.DMA,
            pltpu.SemaphoreType.DMA,
        ],
    )(k_body)
    if has_prev:
        return k(msg, dst, prev)
    return k(msg, dst)


def _stats_body(p0_ref, p1_ref, o_ref, *, n, steps):
    i = pl.program_id(0)

    @pl.when(i == 0)
    def _():
        o_ref[...] = jnp.zeros_like(o_ref)

    sv = p0_ref[0] + p1_ref[0]
    o_ref[0:1, :] += jnp.sum(sv, axis=0, keepdims=True)
    o_ref[1:2, :] += jnp.sum(sv * sv, axis=0, keepdims=True)

    @pl.when(i == steps - 1)
    def _():
        tot = o_ref[0:1, :]
        totsq = o_ref[1:2, :]
        mu = tot / n
        var = totsq / n - mu * mu
        o_ref[0:1, :] = mu
        o_ref[1:2, :] = lax.rsqrt(var + BN_EPS)


def _stats(outp, n):
    _, npad, f = outp.shape
    bn = 1024
    steps = npad // bn
    return pl.pallas_call(
        functools.partial(_stats_body, n=n, steps=steps),
        grid=(steps,),
        in_specs=[
            pl.BlockSpec((1, bn, f), lambda i: (0, i, 0)),
            pl.BlockSpec((1, bn, f), lambda i: (1, i, 0)),
        ],
        out_specs=pl.BlockSpec((8, f), lambda i: (0, 0)),
        out_shape=jax.ShapeDtypeStruct((8, f), F32),
    )(outp, outp)


def _apply_body(x_ref, p0_ref, p1_ref, st_ref, g_ref, b_ref, wd_ref, ws_ref,
                xn_ref, td_ref, ts_ref):
    sv = p0_ref[0] + p1_ref[0]
    mu = st_ref[0:1, :]
    rstd = st_ref[1:2, :]
    bn = g_ref[...] * (sv - mu) * rstd + b_ref[...]
    xn = x_ref[...] + bn
    xn = jnp.where(xn >= 0, xn, NEG_SLOPE * xn)
    xn_ref[...] = xn
    f = xn.shape[1]
    td = jnp.dot(xn, wd_ref[...], preferred_element_type=F32)
    ts = jnp.dot(xn, ws_ref[...], preferred_element_type=F32)
    td_ref[...] = _pack_bf16_pair(td[:, :f], td[:, f:])
    ts_ref[...] = _pack_bf16_pair(ts[:, :f], ts[:, f:])


def _apply_proj(x, outp, st, g, b, wd, ws):
    n, f = x.shape
    f2 = wd.shape[1]
    bnr = 1000
    grid = n // bnr
    return pl.pallas_call(
        _apply_body,
        grid=(grid,),
        in_specs=[
            pl.BlockSpec((bnr, f), lambda i: (i, 0)),
            pl.BlockSpec((1, bnr, f), lambda i: (0, i, 0)),
            pl.BlockSpec((1, bnr, f), lambda i: (1, i, 0)),
            pl.BlockSpec((8, f), lambda i: (0, 0)),
            pl.BlockSpec((1, f), lambda i: (0, 0)),
            pl.BlockSpec((1, f), lambda i: (0, 0)),
            pl.BlockSpec((f, f2), lambda i: (0, 0)),
            pl.BlockSpec((f, f2), lambda i: (0, 0)),
        ],
        out_specs=[
            pl.BlockSpec((bnr, f), lambda i: (i, 0)),
            pl.BlockSpec((bnr, f), lambda i: (i, 0)),
            pl.BlockSpec((bnr, f), lambda i: (i, 0)),
        ],
        out_shape=[
            jax.ShapeDtypeStruct((n, f), F32),
            jax.ShapeDtypeStruct((n, f), jnp.uint32),
            jax.ShapeDtypeStruct((n, f), jnp.uint32),
        ],
    )(x, outp, outp, st, g, b, wd, ws)


def _apply_final_body(x_ref, p0_ref, p1_ref, st_ref, g_ref, b_ref, xn_ref):
    sv = p0_ref[0] + p1_ref[0]
    mu = st_ref[0:1, :]
    rstd = st_ref[1:2, :]
    bn = g_ref[...] * (sv - mu) * rstd + b_ref[...]
    xn = x_ref[...] + bn
    xn_ref[...] = jnp.where(xn >= 0, xn, NEG_SLOPE * xn)


def _apply_final(x, outp, st, g, b):
    n, f = x.shape
    bnr = 1000
    grid = n // bnr
    return pl.pallas_call(
        _apply_final_body,
        grid=(grid,),
        in_specs=[
            pl.BlockSpec((bnr, f), lambda i: (i, 0)),
            pl.BlockSpec((1, bnr, f), lambda i: (0, i, 0)),
            pl.BlockSpec((1, bnr, f), lambda i: (1, i, 0)),
            pl.BlockSpec((8, f), lambda i: (0, 0)),
            pl.BlockSpec((1, f), lambda i: (0, 0)),
            pl.BlockSpec((1, f), lambda i: (0, 0)),
        ],
        out_specs=pl.BlockSpec((bnr, f), lambda i: (i, 0)),
        out_shape=jax.ShapeDtypeStruct((n, f), F32),
    )(x, outp, outp, st, g, b)


def _readout_body(x_ref, w1_ref, b1_ref, w2_ref, b2_ref, bt_ref, o_ref,
                  *, g, bnr, steps):
    i = pl.program_id(0)
    h = jnp.dot(x_ref[...], w1_ref[...], preferred_element_type=F32) + b1_ref[...]
    h = jnp.where(h >= 0, h, NEG_SLOPE * h)
    yv = jnp.sum(h * w2_ref[...], axis=1, keepdims=True) + b2_ref[0:1, 0:1]
    bt = bt_ref[0, 0, :]
    oh = (bt[:, None] == lax.broadcasted_iota(jnp.int32, (bnr, g), 1)).astype(F32)
    hcat = jnp.concatenate(
        [yv, jnp.ones((bnr, 1), F32), jnp.zeros((bnr, 126), F32)], axis=1)
    contrib = lax.dot_general(oh, hcat, (((0,), (0,)), ((), ())),
                              preferred_element_type=F32)

    @pl.when(i == 0)
    def _():
        o_ref[...] = contrib

    @pl.when(i > 0)
    def _():
        o_ref[...] += contrib

    @pl.when(i == steps - 1)
    def _():
        v = o_ref[...]
        sums = v[:, 0:1]
        cnts = v[:, 1:2]
        o_ref[...] = jnp.broadcast_to(sums / jnp.maximum(cnts, 1.0), v.shape)


def _readout(x, w1, b1r, w2r, b2b, batch3d, g):
    n, f = x.shape
    bnr = 1000
    steps = n // bnr
    return pl.pallas_call(
        functools.partial(_readout_body, g=g, bnr=bnr, steps=steps),
        grid=(steps,),
        in_specs=[
            pl.BlockSpec((bnr, f), lambda i: (i, 0)),
            pl.BlockSpec((f, f), lambda i: (0, 0)),
            pl.BlockSpec((1, f), lambda i: (0, 0)),
            pl.BlockSpec((1, f), lambda i: (0, 0)),
            pl.BlockSpec((8, f), lambda i: (0, 0)),
            pl.BlockSpec((1, 1, bnr), lambda i: (i, 0, 0)),
        ],
        out_specs=pl.BlockSpec((g, f), lambda i: (0, 0)),
        out_shape=jax.ShapeDtypeStruct((g, f), F32),
    )(x, w1, b1r, w2r, b2b, batch3d)


def kernel(x, edge_index, edge_attr, batch, Wf, bf, Ws, bs, gamma, beta,
           W1, b1, W2, b2):
    n, f = x.shape
    e = edge_index.shape[1]
    g = 64
    h = W1.shape[1]

    src = edge_index[0]
    dst = edge_index[1]

    wd = jnp.concatenate([Wf[:, :f, :], Ws[:, :f, :]], axis=2)
    wsr = jnp.concatenate([Wf[:, f:2 * f, :], Ws[:, f:2 * f, :]], axis=2)
    wfe = Wf[:, 2 * f:, :]
    wse = Ws[:, 2 * f:, :]
    bf2 = bf[:, None, :]
    bs2 = bs[:, None, :]
    gm2 = gamma[:, None, :]
    bt2 = beta[:, None, :]

    batch3d = batch.reshape(n // 1000, 1, 1000)
    b1r = b1.reshape(1, h)
    w2r = W2.reshape(1, h)
    b2b = jnp.broadcast_to(b2.reshape(1, 1), (8, f))

    xcur = x
    td, ts = _proj(xcur, wd[0], wsr[0])
    for l in range(NUM_LAYERS):
        gd, gs = _gather(td, ts, dst, src)
        msg = _message(gd, gs, edge_attr, wfe[l], wse[l], bf2[l], bs2[l])
        outp = _scatter(msg, dst, n)
        st = _stats(outp, n)
        if l < NUM_LAYERS - 1:
            xcur, td, ts = _apply_proj(xcur, outp, st, gm2[l], bt2[l],
                                       wd[l + 1], wsr[l + 1])
        else:
            xcur = _apply_final(xcur, outp, st, gm2[l], bt2[l])

    pooled = _readout(xcur, W1, b1r, w2r, b2b, batch3d, g)
    return pooled[:, 0:1]


# speedup vs baseline: 1.1274x; 1.1274x over previous
import functools

import jax
import jax.numpy as jnp
from jax import lax
from jax.experimental import pallas as pl
from jax.experimental.pallas import tpu as pltpu
from jax.experimental.pallas import tpu_sc as plsc

F32 = jnp.float32

NC = 2
NS = 16
NW = NC * NS

NUM_LAYERS = 10
NEG_SLOPE = 0.01
BN_EPS = 1e-5


def _pick_chunk(epw: int, cmax: int = 200) -> int:
    for c in range(cmax - cmax % 8, 0, -8):
        if epw % c == 0:
            return c
    raise ValueError(f"no chunk divides {epw}")


def _pack_bf16_pair(hi_f32, lo_f32):
    hb = lax.bitcast_convert_type(hi_f32, jnp.uint32)
    lb = lax.bitcast_convert_type(lo_f32, jnp.uint32)
    hi = (hb + jnp.uint32(0x8000)) & jnp.uint32(0xFFFF0000)
    lo = (lb + jnp.uint32(0x8000)) >> jnp.uint32(16)
    return hi | lo


def _unpack_bf16_pair(p_u32):
    f = lax.bitcast_convert_type(p_u32 & jnp.uint32(0xFFFF0000), F32)
    s = lax.bitcast_convert_type(p_u32 << jnp.uint32(16), F32)
    return f, s


def _proj_body(x_ref, wd_ref, ws_ref, td_ref, ts_ref):
    xv = x_ref[...]
    f = xv.shape[1]
    td = jnp.dot(xv, wd_ref[...], preferred_element_type=F32)
    ts = jnp.dot(xv, ws_ref[...], preferred_element_type=F32)
    td_ref[...] = _pack_bf16_pair(td[:, :f], td[:, f:])
    ts_ref[...] = _pack_bf16_pair(ts[:, :f], ts[:, f:])


def _proj(x, wd, ws):
    n, f = x.shape
    f2 = wd.shape[1]
    bn = 2000
    grid = n // bn
    return pl.pallas_call(
        _proj_body,
        grid=(grid,),
        in_specs=[
            pl.BlockSpec((bn, f), lambda i: (i, 0)),
            pl.BlockSpec((f, f2), lambda i: (0, 0)),
            pl.BlockSpec((f, f2), lambda i: (0, 0)),
        ],
        out_specs=[
            pl.BlockSpec((bn, f), lambda i: (i, 0)),
            pl.BlockSpec((bn, f), lambda i: (i, 0)),
        ],
        out_shape=[
            jax.ShapeDtypeStruct((n, f), jnp.uint32),
            jax.ShapeDtypeStruct((n, f), jnp.uint32),
        ],
    )(x, wd, ws)


def _gather(td, ts, dst, src):
    e = dst.shape[0]
    f = td.shape[1]
    epw = e // NW
    chunk = _pick_chunk(epw)
    nsteps = epw // chunk
    mesh = plsc.VectorSubcoreMesh(
        core_axis_name="c", subcore_axis_name="s", num_cores=NC, num_subcores=NS
    )

    @functools.partial(
        pl.kernel,
        mesh=mesh,
        out_type=[
            jax.ShapeDtypeStruct((e, f), jnp.uint32),
            jax.ShapeDtypeStruct((e, f), jnp.uint32),
        ],
        scratch_types=[
            pltpu.VMEM((chunk,), jnp.int32),
            pltpu.VMEM((chunk,), jnp.int32),
            pltpu.VMEM((chunk,), jnp.int32),
            pltpu.VMEM((chunk,), jnp.int32),
            pltpu.VMEM((chunk, f), jnp.uint32),
            pltpu.VMEM((chunk, f), jnp.uint32),
            pltpu.VMEM((chunk, f), jnp.uint32),
            pltpu.VMEM((chunk, f), jnp.uint32),
            pltpu.SemaphoreType.DMA,
            pltpu.SemaphoreType.DMA,
            pltpu.SemaphoreType.DMA,
            pltpu.SemaphoreType.DMA,
            pltpu.SemaphoreType.DMA,
            pltpu.SemaphoreType.DMA,
        ],
    )
    def k(td_hbm, ts_hbm, dst_hbm, src_hbm, gd_hbm, gs_hbm,
          di0, di1, sri0, sri1, rd0, rd1, rs0, rs1, si0, si1, sg0, sg1,
          sw0, sw1):
        wid = lax.axis_index("s") * NC + lax.axis_index("c")
        tbase = wid * epw
        di = (di0, di1)
        sri = (sri0, sri1)
        rd = (rd0, rd1)
        rs = (rs0, rs1)
        si = (si0, si1)
        sg = (sg0, sg1)
        sw = (sw0, sw1)

        def issue_gathers(b, step):
            pltpu.async_copy(td_hbm.at[di[b]], rd[b], sg[b])
            pltpu.async_copy(ts_hbm.at[sri[b]], rs[b], sg[b])

        def wait_gathers(b, step):
            pltpu.make_async_copy(td_hbm.at[di[b]], rd[b], sg[b]).wait()
            pltpu.make_async_copy(ts_hbm.at[sri[b]], rs[b], sg[b]).wait()

        def issue_idx(b, step):
            base = tbase + step * chunk
            pltpu.async_copy(dst_hbm.at[pl.ds(base, chunk)], di[b], si[b])
            pltpu.async_copy(src_hbm.at[pl.ds(base, chunk)], sri[b], si[b])

        def wait_idx(b, step):
            base = tbase + step * chunk
            pltpu.make_async_copy(
                dst_hbm.at[pl.ds(base, chunk)], di[b], si[b]).wait()
            pltpu.make_async_copy(
                src_hbm.at[pl.ds(base, chunk)], sri[b], si[b]).wait()

        def issue_write(b, step):
            base = tbase + step * chunk
            pltpu.async_copy(rd[b], gd_hbm.at[pl.ds(base, chunk)], sw[b])
            pltpu.async_copy(rs[b], gs_hbm.at[pl.ds(base, chunk)], sw[b])

        def wait_write(b, step):
            base = tbase + step * chunk
            pltpu.make_async_copy(
                rd[b], gd_hbm.at[pl.ds(base, chunk)], sw[b]).wait()
            pltpu.make_async_copy(
                rs[b], gs_hbm.at[pl.ds(base, chunk)], sw[b]).wait()

        pltpu.sync_copy(dst_hbm.at[pl.ds(tbase, chunk)], di0)
        pltpu.sync_copy(src_hbm.at[pl.ds(tbase, chunk)], sri0)
        issue_gathers(0, 0)
        issue_idx(1, 1)

        def step_body(kk, b):
            o = 1 - b
            wait_gathers(b, kk)

            @pl.when(kk >= 1)
            def _():
                wait_write(o, kk - 1)

            @pl.when(kk + 1 < nsteps)
            def _():
                wait_idx(o, kk + 1)
                issue_gathers(o, kk + 1)

                @pl.when(kk + 2 < nsteps)
                def _():
                    issue_idx(b, kk + 2)

            issue_write(b, kk)

        def body(kk, carry):
            @pl.when(kk % 2 == 0)
            def _():
                step_body(kk, 0)

            @pl.when(kk % 2 == 1)
            def _():
                step_body(kk, 1)

            return carry

        lax.fori_loop(0, nsteps, body, 0)
        wait_write((nsteps - 1) % 2, nsteps - 1)

    return k(td, ts, dst, src)


def _message_body(gd_ref, gs_ref, ea_ref, wfe_ref, wse_ref, bf_ref, bs_ref,
                  msg_ref, *, f):
    gd_f, gd_s = _unpack_bf16_pair(gd_ref[...])
    gs_f, gs_s = _unpack_bf16_pair(gs_ref[...])
    eav = ea_ref[...]
    uf = (gd_f + gs_f
          + jnp.dot(eav, wfe_ref[...], preferred_element_type=F32) + bf_ref[...])
    us = (gd_s + gs_s
          + jnp.dot(eav, wse_ref[...], preferred_element_type=F32) + bs_ref[...])
    sig = jax.nn.sigmoid(uf)
    sp = jax.nn.softplus(us)
    msg_ref[...] = sig * sp


def _message(gd, gs, ea, wfe, wse, bfl, bsl):
    e, f = gd.shape
    ed = ea.shape[1]
    be = 2560
    grid = e // be
    return pl.pallas_call(
        functools.partial(_message_body, f=f),
        grid=(grid,),
        in_specs=[
            pl.BlockSpec((be, f), lambda i: (i, 0)),
            pl.BlockSpec((be, f), lambda i: (i, 0)),
            pl.BlockSpec((be, ed), lambda i: (i, 0)),
            pl.BlockSpec((ed, f), lambda i: (0, 0)),
            pl.BlockSpec((ed, f), lambda i: (0, 0)),
            pl.BlockSpec((1, f), lambda i: (0, 0)),
            pl.BlockSpec((1, f), lambda i: (0, 0)),
        ],
        out_specs=pl.BlockSpec((be, f), lambda i: (i, 0)),
        out_shape=jax.ShapeDtypeStruct((e, f), F32),
    )(gd, gs, ea, wfe, wse, bfl, bsl)


def _scatter(msg, dst, n, prev=None):
    e, f = msg.shape
    epw = e // NW
    chunk = _pick_chunk(epw, 125)
    nchunks = epw // chunk
    zc = 80
    assert zc <= chunk
    rows_per_tile = -(-n // NS)
    rows_per_tile = ((rows_per_tile + zc - 1) // zc) * zc
    npad = rows_per_tile * NS
    copies = rows_per_tile // zc
    mesh = plsc.VectorSubcoreMesh(
        core_axis_name="c", subcore_axis_name="s", num_cores=NC, num_subcores=NS
    )
    has_prev = prev is not None

    def k_body(msg_hbm, dst_hbm, *rest):
        if has_prev:
            prev_hbm = rest[0]
            rest = rest[1:]
        (out_hbm, dsti0, dsti1, msgv0, msgv1, acc, sl0, sl1, sa0, sa1) = rest
        c = lax.axis_index("c")
        s = lax.axis_index("s")
        wid = s * NC + c
        tbase = wid * epw
        dsti = (dsti0, dsti1)
        msgv = (msgv0, msgv1)
        sl = (sl0, sl1)
        sa = (sa0, sa1)

        if has_prev:
            pltpu.sync_copy(
                prev_hbm.at[c, pl.ds(s * rows_per_tile, rows_per_tile)],
                acc.at[pl.ds(s * rows_per_tile, rows_per_tile)])
        else:
            def zrow(i, carry):
                for j in range(f // 16):
                    msgv0[i, pl.ds(j * 16, 16)] = jnp.zeros((16,), F32)
                return carry

            lax.fori_loop(0, zc, zrow, 0)
            for j in range(copies):
                pltpu.sync_copy(
                    msgv0.at[pl.ds(0, zc)],
                    acc.at[pl.ds(s * rows_per_tile + j * zc, zc)])
        plsc.subcore_barrier()

        def issue_loads(b, step):
            base = tbase + step * chunk
            pltpu.async_copy(dst_hbm.at[pl.ds(base, chunk)], dsti[b], sl[b])
            pltpu.async_copy(msg_hbm.at[pl.ds(base, chunk)], msgv[b], sl[b])

        def wait_loads(b, step):
            base = tbase + step * chunk
            pltpu.make_async_copy(
                dst_hbm.at[pl.ds(base, chunk)], dsti[b], sl[b]).wait()
            pltpu.make_async_copy(
                msg_hbm.at[pl.ds(base, chunk)], msgv[b], sl[b]).wait()

        issue_loads(0, 0)

        def wait_add(b):
            pltpu.make_async_copy(msgv[b], acc.at[dsti[b]], sa[b]).wait()

        def step_body(kk, b):
            o = 1 - b

            @pl.when(kk >= 1)
            def _():
                wait_add(o)

            @pl.when(kk + 1 < nchunks)
            def _():
                issue_loads(o, kk + 1)

            wait_loads(b, kk)
            pltpu.async_copy(msgv[b], acc.at[dsti[b]], sa[b], add=True)

        def body(kk, carry):
            @pl.when(kk % 2 == 0)
            def _():
                step_body(kk, 0)

            @pl.when(kk % 2 == 1)
            def _():
                step_body(kk, 1)

            return carry

        lax.fori_loop(0, nchunks, body, 0)
        wait_add((nchunks - 1) % 2)
        plsc.subcore_barrier()
        pltpu.sync_copy(
            acc.at[pl.ds(s * rows_per_tile, rows_per_tile)],
            out_hbm.at[c, pl.ds(s * rows_per_tile, rows_per_tile)])

    k = functools.partial(
        pl.kernel,
        mesh=mesh,
        out_type=jax.ShapeDtypeStruct((NC, npad, f), F32),
        scratch_types=[
            pltpu.VMEM((chunk,), jnp.int32),
            pltpu.VMEM((chunk,), jnp.int32),
            pltpu.VMEM((chunk, f), F32),
            pltpu.VMEM((chunk, f), F32),
            pltpu.VMEM_SHARED((npad, f), F32),
            pltpu.SemaphoreType.DMA,
            pltpu.SemaphoreType.DMA,
            pltpu.SemaphoreType.DMA,
            pltpu.SemaphoreType.DMA,
        ],
    )(k_body)
    if has_prev:
        return k(msg, dst, prev)
    return k(msg, dst)


def _stats_body(p0_ref, p1_ref, o_ref, *, n, steps):
    i = pl.program_id(0)

    @pl.when(i == 0)
    def _():
        o_ref[...] = jnp.zeros_like(o_ref)

    sv = p0_ref[0] + p1_ref[0]
    o_ref[0:1, :] += jnp.sum(sv, axis=0, keepdims=True)
    o_ref[1:2, :] += jnp.sum(sv * sv, axis=0, keepdims=True)

    @pl.when(i == steps - 1)
    def _():
        tot = o_ref[0:1, :]
        totsq = o_ref[1:2, :]
        mu = tot / n
        var = totsq / n - mu * mu
        o_ref[0:1, :] = mu
        o_ref[1:2, :] = lax.rsqrt(var + BN_EPS)


def _stats(outp, n):
    _, npad, f = outp.shape
    bn = 1024
    steps = npad // bn
    return pl.pallas_call(
        functools.partial(_stats_body, n=n, steps=steps),
        grid=(steps,),
        in_specs=[
            pl.BlockSpec((1, bn, f), lambda i: (0, i, 0)),
            pl.BlockSpec((1, bn, f), lambda i: (1, i, 0)),
        ],
        out_specs=pl.BlockSpec((8, f), lambda i: (0, 0)),
        out_shape=jax.ShapeDtypeStruct((8, f), F32),
    )(outp, outp)


def _apply_body(x_ref, p0_ref, p1_ref, st_ref, g_ref, b_ref, wd_ref, ws_ref,
                xn_ref, td_ref, ts_ref):
    sv = p0_ref[0] + p1_ref[0]
    mu = st_ref[0:1, :]
    rstd = st_ref[1:2, :]
    bn = g_ref[...] * (sv - mu) * rstd + b_ref[...]
    xn = x_ref[...] + bn
    xn = jnp.where(xn >= 0, xn, NEG_SLOPE * xn)
    xn_ref[...] = xn
    f = xn.shape[1]
    td = jnp.dot(xn, wd_ref[...], preferred_element_type=F32)
    ts = jnp.dot(xn, ws_ref[...], preferred_element_type=F32)
    td_ref[...] = _pack_bf16_pair(td[:, :f], td[:, f:])
    ts_ref[...] = _pack_bf16_pair(ts[:, :f], ts[:, f:])


def _apply_proj(x, outp, st, g, b, wd, ws):
    n, f = x.shape
    f2 = wd.shape[1]
    bnr = 1000
    grid = n // bnr
    return pl.pallas_call(
        _apply_body,
        grid=(grid,),
        in_specs=[
            pl.BlockSpec((bnr, f), lambda i: (i, 0)),
            pl.BlockSpec((1, bnr, f), lambda i: (0, i, 0)),
            pl.BlockSpec((1, bnr, f), lambda i: (1, i, 0)),
            pl.BlockSpec((8, f), lambda i: (0, 0)),
            pl.BlockSpec((1, f), lambda i: (0, 0)),
            pl.BlockSpec((1, f), lambda i: (0, 0)),
            pl.BlockSpec((f, f2), lambda i: (0, 0)),
            pl.BlockSpec((f, f2), lambda i: (0, 0)),
        ],
        out_specs=[
            pl.BlockSpec((bnr, f), lambda i: (i, 0)),
            pl.BlockSpec((bnr, f), lambda i: (i, 0)),
            pl.BlockSpec((bnr, f), lambda i: (i, 0)),
        ],
        out_shape=[
            jax.ShapeDtypeStruct((n, f), F32),
            jax.ShapeDtypeStruct((n, f), jnp.uint32),
            jax.ShapeDtypeStruct((n, f), jnp.uint32),
        ],
    )(x, outp, outp, st, g, b, wd, ws)


def _apply_final_body(x_ref, p0_ref, p1_ref, st_ref, g_ref, b_ref, xn_ref):
    sv = p0_ref[0] + p1_ref[0]
    mu = st_ref[0:1, :]
    rstd = st_ref[1:2, :]
    bn = g_ref[...] * (sv - mu) * rstd + b_ref[...]
    xn = x_ref[...] + bn
    xn_ref[...] = jnp.where(xn >= 0, xn, NEG_SLOPE * xn)


def _apply_final(x, outp, st, g, b):
    n, f = x.shape
    bnr = 1000
    grid = n // bnr
    return pl.pallas_call(
        _apply_final_body,
        grid=(grid,),
        in_specs=[
            pl.BlockSpec((bnr, f), lambda i: (i, 0)),
            pl.BlockSpec((1, bnr, f), lambda i: (0, i, 0)),
            pl.BlockSpec((1, bnr, f), lambda i: (1, i, 0)),
            pl.BlockSpec((8, f), lambda i: (0, 0)),
            pl.BlockSpec((1, f), lambda i: (0, 0)),
            pl.BlockSpec((1, f), lambda i: (0, 0)),
        ],
        out_specs=pl.BlockSpec((bnr, f), lambda i: (i, 0)),
        out_shape=jax.ShapeDtypeStruct((n, f), F32),
    )(x, outp, outp, st, g, b)


def _readout_body(x_ref, w1_ref, b1_ref, w2_ref, b2_ref, bt_ref, o_ref,
                  *, g, bnr, steps):
    i = pl.program_id(0)
    h = jnp.dot(x_ref[...], w1_ref[...], preferred_element_type=F32) + b1_ref[...]
    h = jnp.where(h >= 0, h, NEG_SLOPE * h)
    yv = jnp.sum(h * w2_ref[...], axis=1, keepdims=True) + b2_ref[0:1, 0:1]
    bt = bt_ref[0, 0, :]
    oh = (bt[:, None] == lax.broadcasted_iota(jnp.int32, (bnr, g), 1)).astype(F32)
    hcat = jnp.concatenate(
        [yv, jnp.ones((bnr, 1), F32), jnp.zeros((bnr, 126), F32)], axis=1)
    contrib = lax.dot_general(oh, hcat, (((0,), (0,)), ((), ())),
                              preferred_element_type=F32)

    @pl.when(i == 0)
    def _():
        o_ref[...] = contrib

    @pl.when(i > 0)
    def _():
        o_ref[...] += contrib

    @pl.when(i == steps - 1)
    def _():
        v = o_ref[...]
        sums = v[:, 0:1]
        cnts = v[:, 1:2]
        o_ref[...] = jnp.broadcast_to(sums / jnp.maximum(cnts, 1.0), v.shape)


def _readout(x, w1, b1r, w2r, b2b, batch3d, g):
    n, f = x.shape
    bnr = 1000
    steps = n // bnr
    return pl.pallas_call(
        functools.partial(_readout_body, g=g, bnr=bnr, steps=steps),
        grid=(steps,),
        in_specs=[
            pl.BlockSpec((bnr, f), lambda i: (i, 0)),
            pl.BlockSpec((f, f), lambda i: (0, 0)),
            pl.BlockSpec((1, f), lambda i: (0, 0)),
            pl.BlockSpec((1, f), lambda i: (0, 0)),
            pl.BlockSpec((8, f), lambda i: (0, 0)),
            pl.BlockSpec((1, 1, bnr), lambda i: (i, 0, 0)),
        ],
        out_specs=pl.BlockSpec((g, f), lambda i: (0, 0)),
        out_shape=jax.ShapeDtypeStruct((g, f), F32),
    )(x, w1, b1r, w2r, b2b, batch3d)


def kernel(x, edge_index, edge_attr, batch, Wf, bf, Ws, bs, gamma, beta,
           W1, b1, W2, b2):
    n, f = x.shape
    e = edge_index.shape[1]
    g = 64
    h = W1.shape[1]

    src = edge_index[0]
    dst = edge_index[1]

    wd = jnp.concatenate([Wf[:, :f, :], Ws[:, :f, :]], axis=2)
    wsr = jnp.concatenate([Wf[:, f:2 * f, :], Ws[:, f:2 * f, :]], axis=2)
    wfe = Wf[:, 2 * f:, :]
    wse = Ws[:, 2 * f:, :]
    bf2 = bf[:, None, :]
    bs2 = bs[:, None, :]
    gm2 = gamma[:, None, :]
    bt2 = beta[:, None, :]

    batch3d = batch.reshape(n // 1000, 1, 1000)
    b1r = b1.reshape(1, h)
    w2r = W2.reshape(1, h)
    b2b = jnp.broadcast_to(b2.reshape(1, 1), (8, f))

    ea = e * 3 // 5
    dstA, dstB = dst[:ea], dst[ea:]
    srcA, srcB = src[:ea], src[ea:]
    eaA, eaB = edge_attr[:ea], edge_attr[ea:]

    xcur = x
    td, ts = _proj(xcur, wd[0], wsr[0])
    for l in range(NUM_LAYERS):
        gdA, gsA = _gather(td, ts, dstA, srcA)
        gdB, gsB = _gather(td, ts, dstB, srcB)
        msgA = _message(gdA, gsA, eaA, wfe[l], wse[l], bf2[l], bs2[l])
        msgB = _message(gdB, gsB, eaB, wfe[l], wse[l], bf2[l], bs2[l])
        pA = _scatter(msgA, dstA, n)
        outp = _scatter(msgB, dstB, n, prev=pA)
        st = _stats(outp, n)
        if l < NUM_LAYERS - 1:
            xcur, td, ts = _apply_proj(xcur, outp, st, gm2[l], bt2[l],
                                       wd[l + 1], wsr[l + 1])
        else:
            xcur = _apply_final(xcur, outp, st, gm2[l], bt2[l])

    pooled = _readout(xcur, W1, b1r, w2r, b2b, batch3d, g)
    return pooled[:, 0:1]
